# baseline (speedup 1.0000x reference)
import functools

import jax
import jax.numpy as jnp
from jax import lax
from jax.experimental import pallas as pl
from jax.experimental.pallas import tpu as pltpu
from jax.experimental.pallas import tpu_sc as plsc

DIM = 256
N = 8192
C = 8192
TN = 256
TC = 1024
N_TILES = N // TN
C_TILES = C // TC


def _row_sumsq_t(v):
    a = v * v
    p = a[:, :128] + a[:, 128:]
    q = jnp.transpose(p)
    acc = q[0:8, :]
    for i in range(1, 16):
        acc = acc + q[8 * i:8 * i + 8, :]
    b = acc[0:4, :] + acc[4:8, :]
    b = b[0:2, :] + b[2:4, :]
    return b[0:1, :] + b[1:2, :]


def _norms_body(x_ref, e_ref, x2_ref, e2_ref):
    x2_ref[...] = _row_sumsq_t(x_ref[...])
    e2_ref[...] = _row_sumsq_t(e_ref[...])


_norms_call = pl.pallas_call(
    _norms_body,
    grid=(8,),
    in_specs=[
        pl.BlockSpec((N // 8, DIM), lambda i: (i, 0)),
        pl.BlockSpec((C // 8, DIM), lambda i: (i, 0)),
    ],
    out_specs=[
        pl.BlockSpec((1, N // 8), lambda i: (0, i)),
        pl.BlockSpec((1, C // 8), lambda i: (0, i)),
    ],
    out_shape=[
        jax.ShapeDtypeStruct((1, N), jnp.float32),
        jax.ShapeDtypeStruct((1, C), jnp.float32),
    ],
)


def _dist_body(x_ref, e2x_ref, x2_ref, e2_ref, iota_ref, dist_ref, ind_ref):
    x = x_ref[...]
    inner2 = lax.dot_general(x, e2x_ref[...], (((1,), (1,)), ((), ())),
                             preferred_element_type=jnp.float32)
    x2 = jnp.transpose(x2_ref[...])
    d2 = jnp.clip((x2 + e2_ref[...]) - inner2, 0.0, None)
    dist = -jnp.sqrt(d2)
    dist_ref[...] = dist

    m = jnp.max(dist, axis=1, keepdims=True)
    lanes = jnp.broadcast_to(iota_ref[...], (TN, C))
    cand = (m - dist) * jnp.float32(3e38) + lanes
    best = jnp.min(cand, axis=1, keepdims=True)
    ind_ref[...] = best.astype(jnp.int32)


_dist_call = pl.pallas_call(
    _dist_body,
    grid=(N_TILES,),
    in_specs=[
        pl.BlockSpec((TN, DIM), lambda i: (i, 0)),
        pl.BlockSpec((C, DIM), lambda i: (0, 0)),
        pl.BlockSpec((1, TN), lambda i: (0, i)),
        pl.BlockSpec((1, C), lambda i: (0, 0)),
        pl.BlockSpec((1, C), lambda i: (0, 0)),
    ],
    out_specs=[
        pl.BlockSpec((TN, C), lambda i: (i, 0)),
        pl.BlockSpec((TN, 1), lambda i: (i, 0)),
    ],
    out_shape=[
        jax.ShapeDtypeStruct((N, C), jnp.float32),
        jax.ShapeDtypeStruct((N, 1), jnp.int32),
    ],
)


_NC = 2
_NS = 16
_NW = _NC * _NS
_BPW = N // _NW


@functools.cache
def _gather_rows_call():
    @functools.partial(
        pl.kernel,
        out_type=jax.ShapeDtypeStruct((N, DIM), jnp.float32),
        mesh=plsc.VectorSubcoreMesh(core_axis_name="c", subcore_axis_name="s"),
        scratch_types=[
            pltpu.VMEM((_BPW,), jnp.int32),
            pltpu.VMEM((_BPW, DIM), jnp.float32),
            pltpu.SemaphoreType.DMA,
        ],
    )
    def _gather_rows(table_hbm, idx_hbm, out_hbm, idx_v, rows_v, sem):
        wid = lax.axis_index("s") * _NC + lax.axis_index("c")
        base = wid * _BPW
        pltpu.sync_copy(idx_hbm.at[pl.ds(base, _BPW)], idx_v)
        pltpu.async_copy(table_hbm.at[idx_v], rows_v, sem).wait()
        pltpu.sync_copy(rows_v, out_hbm.at[pl.ds(base, _BPW)])

    return _gather_rows


def kernel(x, embeddings):
    orig_shape = x.shape
    xf = x.reshape(N, DIM)
    table = embeddings.reshape(C, DIM)

    x2, e2 = _norms_call(xf, table)
    iota_row = jnp.arange(C, dtype=jnp.float32).reshape(1, C)
    dist, ind = _dist_call(xf, table * 2.0, x2, e2, iota_row)
    idx_flat = ind.reshape(N)

    quantize = _gather_rows_call()(table, idx_flat)

    return (quantize.reshape(orig_shape),
            idx_flat.reshape(orig_shape[:-1]),
            dist[None, ...])


# speedup vs baseline: 1.0422x; 1.0422x over previous
import functools

import jax
import jax.numpy as jnp
from jax import lax
from jax.experimental import pallas as pl
from jax.experimental.pallas import tpu as pltpu
from jax.experimental.pallas import tpu_sc as plsc

DIM = 256
N = 8192
C = 8192
TN = 256
TC = 1024
N_TILES = N // TN
C_TILES = C // TC


def _row_sumsq_t(v):
    a = v * v
    p = a[:, :128] + a[:, 128:]
    q = jnp.transpose(p)
    acc = q[0:8, :]
    for i in range(1, 16):
        acc = acc + q[8 * i:8 * i + 8, :]
    b = acc[0:4, :] + acc[4:8, :]
    b = b[0:2, :] + b[2:4, :]
    return b[0:1, :] + b[1:2, :]


def _norms_body(x_ref, e_ref, x2_ref, e2_ref):
    x2_ref[...] = _row_sumsq_t(x_ref[...])
    e2_ref[...] = _row_sumsq_t(e_ref[...])


_norms_call = pl.pallas_call(
    _norms_body,
    grid=(8,),
    in_specs=[
        pl.BlockSpec((N // 8, DIM), lambda i: (i, 0)),
        pl.BlockSpec((C // 8, DIM), lambda i: (i, 0)),
    ],
    out_specs=[
        pl.BlockSpec((1, N // 8), lambda i: (0, i)),
        pl.BlockSpec((1, C // 8), lambda i: (0, i)),
    ],
    out_shape=[
        jax.ShapeDtypeStruct((1, N), jnp.float32),
        jax.ShapeDtypeStruct((1, C), jnp.float32),
    ],
)


def _dist_body(x_ref, e2x_ref, x2_ref, e2_ref, iota_ref, dist_ref, ind_ref):
    x = x_ref[...]
    inner2 = lax.dot_general(x, e2x_ref[...], (((1,), (1,)), ((), ())),
                             preferred_element_type=jnp.float32)
    x2 = jnp.transpose(x2_ref[...])
    e2 = e2_ref[...]

    W = 128
    colmax = None
    colk = None
    for k in range(C // W):
        sl = slice(k * W, (k + 1) * W)
        d2 = jnp.clip((x2 + e2[:, sl]) - inner2[:, sl], 0.0, None)
        dk = -jnp.sqrt(d2)
        dist_ref[:, sl] = dk
        if k == 0:
            colmax = dk
            colk = jnp.zeros((TN, W), jnp.float32)
        else:
            newer = dk > colmax
            colk = jnp.where(newer, jnp.float32(k), colk)
            colmax = jnp.maximum(colmax, dk)

    m = jnp.max(colmax, axis=1, keepdims=True)
    lane = jnp.broadcast_to(iota_ref[:, :W], (TN, W))
    gidx = colk * jnp.float32(W) + lane
    cand = (m - colmax) * jnp.float32(3e38) + gidx
    best = jnp.min(cand, axis=1, keepdims=True)
    ind_ref[...] = best.astype(jnp.int32)


_dist_call = pl.pallas_call(
    _dist_body,
    grid=(N_TILES,),
    in_specs=[
        pl.BlockSpec((TN, DIM), lambda i: (i, 0)),
        pl.BlockSpec((C, DIM), lambda i: (0, 0)),
        pl.BlockSpec((1, TN), lambda i: (0, i)),
        pl.BlockSpec((1, C), lambda i: (0, 0)),
        pl.BlockSpec((1, C), lambda i: (0, 0)),
    ],
    out_specs=[
        pl.BlockSpec((TN, C), lambda i: (i, 0)),
        pl.BlockSpec((TN, 1), lambda i: (i, 0)),
    ],
    out_shape=[
        jax.ShapeDtypeStruct((N, C), jnp.float32),
        jax.ShapeDtypeStruct((N, 1), jnp.int32),
    ],
)


_NC = 2
_NS = 16
_NW = _NC * _NS
_BPW = N // _NW


@functools.cache
def _gather_rows_call():
    @functools.partial(
        pl.kernel,
        out_type=jax.ShapeDtypeStruct((N, DIM), jnp.float32),
        mesh=plsc.VectorSubcoreMesh(core_axis_name="c", subcore_axis_name="s"),
        scratch_types=[
            pltpu.VMEM((_BPW,), jnp.int32),
            pltpu.VMEM((_BPW, DIM), jnp.float32),
            pltpu.SemaphoreType.DMA,
        ],
    )
    def _gather_rows(table_hbm, idx_hbm, out_hbm, idx_v, rows_v, sem):
        wid = lax.axis_index("s") * _NC + lax.axis_index("c")
        base = wid * _BPW
        pltpu.sync_copy(idx_hbm.at[pl.ds(base, _BPW)], idx_v)
        pltpu.async_copy(table_hbm.at[idx_v], rows_v, sem).wait()
        pltpu.sync_copy(rows_v, out_hbm.at[pl.ds(base, _BPW)])

    return _gather_rows


def kernel(x, embeddings):
    orig_shape = x.shape
    xf = x.reshape(N, DIM)
    table = embeddings.reshape(C, DIM)

    x2, e2 = _norms_call(xf, table)
    iota_row = jnp.arange(C, dtype=jnp.float32).reshape(1, C)
    dist, ind = _dist_call(xf, table * 2.0, x2, e2, iota_row)
    idx_flat = ind.reshape(N)

    quantize = _gather_rows_call()(table, idx_flat)

    return (quantize.reshape(orig_shape),
            idx_flat.reshape(orig_shape[:-1]),
            dist[None, ...])


# speedup vs baseline: 1.2313x; 1.1814x over previous
import functools

import jax
import jax.numpy as jnp
from jax import lax
from jax.experimental import pallas as pl
from jax.experimental.pallas import tpu as pltpu
from jax.experimental.pallas import tpu_sc as plsc

DIM = 256
N = 8192
C = 8192
TN = 256
TC = 1024
N_TILES = N // TN
C_TILES = C // TC


def _row_sumsq_t(v):
    a = v * v
    p = a[:, :128] + a[:, 128:]
    q = jnp.transpose(p)
    acc = q[0:8, :]
    for i in range(1, 16):
        acc = acc + q[8 * i:8 * i + 8, :]
    b = acc[0:4, :] + acc[4:8, :]
    b = b[0:2, :] + b[2:4, :]
    return b[0:1, :] + b[1:2, :]


def _norms_body(x_ref, e_ref, x2_ref, e2_ref):
    x2_ref[...] = _row_sumsq_t(x_ref[...])
    e2_ref[...] = _row_sumsq_t(e_ref[...])


_norms_call = pl.pallas_call(
    _norms_body,
    grid=(8,),
    in_specs=[
        pl.BlockSpec((N // 8, DIM), lambda i: (i, 0)),
        pl.BlockSpec((C // 8, DIM), lambda i: (i, 0)),
    ],
    out_specs=[
        pl.BlockSpec((1, N // 8), lambda i: (0, i)),
        pl.BlockSpec((1, C // 8), lambda i: (0, i)),
    ],
    out_shape=[
        jax.ShapeDtypeStruct((1, N), jnp.float32),
        jax.ShapeDtypeStruct((1, C), jnp.float32),
    ],
)


def _dist_body(x_ref, e2x_ref, x2_ref, e2_ref, iota_ref, dist_ref, ind_ref):
    x = x_ref[...]
    inner2 = lax.dot_general(x, e2x_ref[...], (((1,), (1,)), ((), ())),
                             preferred_element_type=jnp.float32)
    x2 = jnp.transpose(x2_ref[...])
    e2 = e2_ref[...]

    W = 128
    colmax = None
    colk = None
    for k in range(C // W):
        sl = slice(k * W, (k + 1) * W)
        d2 = jnp.clip((x2 + e2[:, sl]) - inner2[:, sl], 0.0, None)
        y = d2 * lax.rsqrt(d2)
        dk = -jnp.where(d2 == 0.0, jnp.float32(0.0), y)
        dist_ref[:, sl] = dk
        if k == 0:
            colmax = dk
            colk = jnp.zeros((TN, W), jnp.float32)
        else:
            newer = dk > colmax
            colk = jnp.where(newer, jnp.float32(k), colk)
            colmax = jnp.maximum(colmax, dk)

    m = jnp.max(colmax, axis=1, keepdims=True)
    lane = jnp.broadcast_to(iota_ref[:, :W], (TN, W))
    gidx = colk * jnp.float32(W) + lane
    cand = (m - colmax) * jnp.float32(3e38) + gidx
    best = jnp.min(cand, axis=1, keepdims=True)
    ind_ref[...] = best.astype(jnp.int32)


_dist_call = pl.pallas_call(
    _dist_body,
    grid=(N_TILES,),
    in_specs=[
        pl.BlockSpec((TN, DIM), lambda i: (i, 0)),
        pl.BlockSpec((C, DIM), lambda i: (0, 0)),
        pl.BlockSpec((1, TN), lambda i: (0, i)),
        pl.BlockSpec((1, C), lambda i: (0, 0)),
        pl.BlockSpec((1, C), lambda i: (0, 0)),
    ],
    out_specs=[
        pl.BlockSpec((TN, C), lambda i: (i, 0)),
        pl.BlockSpec((TN, 1), lambda i: (i, 0)),
    ],
    out_shape=[
        jax.ShapeDtypeStruct((N, C), jnp.float32),
        jax.ShapeDtypeStruct((N, 1), jnp.int32),
    ],
)


_NC = 2
_NS = 16
_NW = _NC * _NS
_BPW = N // _NW


@functools.cache
def _gather_rows_call():
    @functools.partial(
        pl.kernel,
        out_type=jax.ShapeDtypeStruct((N, DIM), jnp.float32),
        mesh=plsc.VectorSubcoreMesh(core_axis_name="c", subcore_axis_name="s"),
        scratch_types=[
            pltpu.VMEM((_BPW,), jnp.int32),
            pltpu.VMEM((_BPW, DIM), jnp.float32),
            pltpu.SemaphoreType.DMA,
        ],
    )
    def _gather_rows(table_hbm, idx_hbm, out_hbm, idx_v, rows_v, sem):
        wid = lax.axis_index("s") * _NC + lax.axis_index("c")
        base = wid * _BPW
        pltpu.sync_copy(idx_hbm.at[pl.ds(base, _BPW)], idx_v)
        pltpu.async_copy(table_hbm.at[idx_v], rows_v, sem).wait()
        pltpu.sync_copy(rows_v, out_hbm.at[pl.ds(base, _BPW)])

    return _gather_rows


def kernel(x, embeddings):
    orig_shape = x.shape
    xf = x.reshape(N, DIM)
    table = embeddings.reshape(C, DIM)

    x2, e2 = _norms_call(xf, table)
    iota_row = jnp.arange(C, dtype=jnp.float32).reshape(1, C)
    dist, ind = _dist_call(xf, table * 2.0, x2, e2, iota_row)
    idx_flat = ind.reshape(N)

    quantize = _gather_rows_call()(table, idx_flat)

    return (quantize.reshape(orig_shape),
            idx_flat.reshape(orig_shape[:-1]),
            dist[None, ...])


# speedup vs baseline: 1.2654x; 1.0277x over previous
import functools

import jax
import jax.numpy as jnp
from jax import lax
from jax.experimental import pallas as pl
from jax.experimental.pallas import tpu as pltpu
from jax.experimental.pallas import tpu_sc as plsc

DIM = 256
N = 8192
C = 8192
TN = 256
TC = 1024
N_TILES = N // TN
C_TILES = C // TC


def _row_sumsq_t(v):
    a = v * v
    p = a[:, :128] + a[:, 128:]
    q = jnp.transpose(p)
    acc = q[0:8, :]
    for i in range(1, 16):
        acc = acc + q[8 * i:8 * i + 8, :]
    b = acc[0:4, :] + acc[4:8, :]
    b = b[0:2, :] + b[2:4, :]
    return b[0:1, :] + b[1:2, :]


def _norms_body(e_ref, e2_ref, e2x_ref):
    e = e_ref[...]
    e2_ref[...] = _row_sumsq_t(e)
    e2x_ref[...] = e + e


_norms_call = pl.pallas_call(
    _norms_body,
    grid=(8,),
    in_specs=[
        pl.BlockSpec((C // 8, DIM), lambda i: (i, 0)),
    ],
    out_specs=[
        pl.BlockSpec((1, C // 8), lambda i: (0, i)),
        pl.BlockSpec((C // 8, DIM), lambda i: (i, 0)),
    ],
    out_shape=[
        jax.ShapeDtypeStruct((1, C), jnp.float32),
        jax.ShapeDtypeStruct((C, DIM), jnp.float32),
    ],
)


def _dist_body(x_ref, e2x_ref, e2_ref, iota_ref, dist_ref, ind_ref):
    x = x_ref[...]
    inner2 = lax.dot_general(x, e2x_ref[...], (((1,), (1,)), ((), ())),
                             preferred_element_type=jnp.float32)
    x2 = jnp.transpose(_row_sumsq_t(x))
    e2 = e2_ref[...]

    W = 128
    colmax = None
    colk = None
    for k in range(C // W):
        sl = slice(k * W, (k + 1) * W)
        d2 = jnp.clip((x2 + e2[:, sl]) - inner2[:, sl], 0.0, None)
        y = d2 * lax.rsqrt(d2)
        dk = -jnp.where(d2 == 0.0, jnp.float32(0.0), y)
        dist_ref[:, sl] = dk
        if k == 0:
            colmax = dk
            colk = jnp.zeros((TN, W), jnp.float32)
        else:
            newer = dk > colmax
            colk = jnp.where(newer, jnp.float32(k), colk)
            colmax = jnp.maximum(colmax, dk)

    m = jnp.max(colmax, axis=1, keepdims=True)
    lane = jnp.broadcast_to(iota_ref[:, :W], (TN, W))
    gidx = colk * jnp.float32(W) + lane
    cand = (m - colmax) * jnp.float32(3e38) + gidx
    best = jnp.min(cand, axis=1, keepdims=True)
    ind_ref[...] = best.astype(jnp.int32)


_dist_call = pl.pallas_call(
    _dist_body,
    grid=(N_TILES,),
    in_specs=[
        pl.BlockSpec((TN, DIM), lambda i: (i, 0)),
        pl.BlockSpec((C, DIM), lambda i: (0, 0)),
        pl.BlockSpec((1, C), lambda i: (0, 0)),
        pl.BlockSpec((1, C), lambda i: (0, 0)),
    ],
    out_specs=[
        pl.BlockSpec((TN, C), lambda i: (i, 0)),
        pl.BlockSpec((TN, 1), lambda i: (i, 0)),
    ],
    out_shape=[
        jax.ShapeDtypeStruct((N, C), jnp.float32),
        jax.ShapeDtypeStruct((N, 1), jnp.int32),
    ],
)


_NC = 2
_NS = 16
_NW = _NC * _NS
_BPW = N // _NW


@functools.cache
def _gather_rows_call():
    @functools.partial(
        pl.kernel,
        out_type=jax.ShapeDtypeStruct((N, DIM), jnp.float32),
        mesh=plsc.VectorSubcoreMesh(core_axis_name="c", subcore_axis_name="s"),
        scratch_types=[
            pltpu.VMEM((_BPW,), jnp.int32),
            pltpu.VMEM((_BPW, DIM), jnp.float32),
            pltpu.SemaphoreType.DMA,
        ],
    )
    def _gather_rows(table_hbm, idx_hbm, out_hbm, idx_v, rows_v, sem):
        wid = lax.axis_index("s") * _NC + lax.axis_index("c")
        base = wid * _BPW
        pltpu.sync_copy(idx_hbm.at[pl.ds(base, _BPW)], idx_v)
        pltpu.async_copy(table_hbm.at[idx_v], rows_v, sem).wait()
        pltpu.sync_copy(rows_v, out_hbm.at[pl.ds(base, _BPW)])

    return _gather_rows


def kernel(x, embeddings):
    orig_shape = x.shape
    xf = x.reshape(N, DIM)
    table = embeddings.reshape(C, DIM)

    e2, e2x = _norms_call(table)
    iota_row = jnp.arange(C, dtype=jnp.float32).reshape(1, C)
    dist, ind = _dist_call(xf, e2x, e2, iota_row)
    idx_flat = ind.reshape(N)

    quantize = _gather_rows_call()(table, idx_flat)

    return (quantize.reshape(orig_shape),
            idx_flat.reshape(orig_shape[:-1]),
            dist[None, ...])


# speedup vs baseline: 1.2715x; 1.0048x over previous
import functools

import jax
import jax.numpy as jnp
from jax import lax
from jax.experimental import pallas as pl
from jax.experimental.pallas import tpu as pltpu
from jax.experimental.pallas import tpu_sc as plsc

DIM = 256
N = 8192
C = 8192
TN = 512
TC = 1024
N_TILES = N // TN
C_TILES = C // TC


def _row_sumsq_t(v):
    a = v * v
    p = a[:, :128] + a[:, 128:]
    q = jnp.transpose(p)
    acc = q[0:8, :]
    for i in range(1, 16):
        acc = acc + q[8 * i:8 * i + 8, :]
    b = acc[0:4, :] + acc[4:8, :]
    b = b[0:2, :] + b[2:4, :]
    return b[0:1, :] + b[1:2, :]


def _norms_body(e_ref, e2_ref, e2x_ref):
    e = e_ref[...]
    e2_ref[...] = _row_sumsq_t(e)
    e2x_ref[...] = e + e


_norms_call = pl.pallas_call(
    _norms_body,
    grid=(8,),
    in_specs=[
        pl.BlockSpec((C // 8, DIM), lambda i: (i, 0)),
    ],
    out_specs=[
        pl.BlockSpec((1, C // 8), lambda i: (0, i)),
        pl.BlockSpec((C // 8, DIM), lambda i: (i, 0)),
    ],
    out_shape=[
        jax.ShapeDtypeStruct((1, C), jnp.float32),
        jax.ShapeDtypeStruct((C, DIM), jnp.float32),
    ],
)


def _dist_body(x_ref, e2x_ref, e2_ref, iota_ref, dist_ref, ind_ref):
    x = x_ref[...]
    inner2 = lax.dot_general(x, e2x_ref[...], (((1,), (1,)), ((), ())),
                             preferred_element_type=jnp.float32)
    x2 = jnp.transpose(_row_sumsq_t(x))
    e2 = e2_ref[...]

    W = 128
    colmax = None
    colk = None
    for k in range(C // W):
        sl = slice(k * W, (k + 1) * W)
        d2 = jnp.clip((x2 + e2[:, sl]) - inner2[:, sl], 0.0, None)
        y = d2 * lax.rsqrt(d2)
        dk = -jnp.where(d2 == 0.0, jnp.float32(0.0), y)
        dist_ref[:, sl] = dk
        if k == 0:
            colmax = dk
            colk = jnp.zeros((TN, W), jnp.float32)
        else:
            newer = dk > colmax
            colk = jnp.where(newer, jnp.float32(k), colk)
            colmax = jnp.maximum(colmax, dk)

    m = jnp.max(colmax, axis=1, keepdims=True)
    lane = jnp.broadcast_to(iota_ref[:, :W], (TN, W))
    gidx = colk * jnp.float32(W) + lane
    cand = (m - colmax) * jnp.float32(3e38) + gidx
    best = jnp.min(cand, axis=1, keepdims=True)
    ind_ref[...] = best.astype(jnp.int32)


_dist_call = pl.pallas_call(
    _dist_body,
    grid=(N_TILES,),
    in_specs=[
        pl.BlockSpec((TN, DIM), lambda i: (i, 0)),
        pl.BlockSpec((C, DIM), lambda i: (0, 0)),
        pl.BlockSpec((1, C), lambda i: (0, 0)),
        pl.BlockSpec((1, C), lambda i: (0, 0)),
    ],
    out_specs=[
        pl.BlockSpec((TN, C), lambda i: (i, 0)),
        pl.BlockSpec((TN, 1), lambda i: (i, 0)),
    ],
    out_shape=[
        jax.ShapeDtypeStruct((N, C), jnp.float32),
        jax.ShapeDtypeStruct((N, 1), jnp.int32),
    ],
)


_NC = 2
_NS = 16
_NW = _NC * _NS
_BPW = N // _NW


@functools.cache
def _gather_rows_call():
    @functools.partial(
        pl.kernel,
        out_type=jax.ShapeDtypeStruct((N, DIM), jnp.float32),
        mesh=plsc.VectorSubcoreMesh(core_axis_name="c", subcore_axis_name="s"),
        scratch_types=[
            pltpu.VMEM((_BPW,), jnp.int32),
            pltpu.VMEM((_BPW, DIM), jnp.float32),
            pltpu.SemaphoreType.DMA,
        ],
    )
    def _gather_rows(table_hbm, idx_hbm, out_hbm, idx_v, rows_v, sem):
        wid = lax.axis_index("s") * _NC + lax.axis_index("c")
        base = wid * _BPW
        pltpu.sync_copy(idx_hbm.at[pl.ds(base, _BPW)], idx_v)
        pltpu.async_copy(table_hbm.at[idx_v], rows_v, sem).wait()
        pltpu.sync_copy(rows_v, out_hbm.at[pl.ds(base, _BPW)])

    return _gather_rows


def kernel(x, embeddings):
    orig_shape = x.shape
    xf = x.reshape(N, DIM)
    table = embeddings.reshape(C, DIM)

    e2, e2x = _norms_call(table)
    iota_row = jnp.arange(C, dtype=jnp.float32).reshape(1, C)
    dist, ind = _dist_call(xf, e2x, e2, iota_row)
    idx_flat = ind.reshape(N)

    quantize = _gather_rows_call()(table, idx_flat)

    return (quantize.reshape(orig_shape),
            idx_flat.reshape(orig_shape[:-1]),
            dist[None, ...])
